# baseline (speedup 1.0000x reference)
import functools

import jax
import jax.numpy as jnp
from jax import lax
from jax.experimental import pallas as pl
from jax.experimental.pallas import tpu as pltpu
from jax.experimental.pallas import tpu_sc as plsc

VOCAB = 100000
D = 128
H = D // 2
MAXLEN = 256
BATCH = 4096
SEQ = 200

NUM_WORKERS = 32
CHUNK = 128
NBUF = 4

K_SLICES = 4
SLICE_B = BATCH // K_SLICES
SLICE_TOK = SLICE_B * SEQ
TOK_PER_W = SLICE_TOK // NUM_WORKERS
NCHUNKS = TOK_PER_W // CHUNK

TC_SEQS = 8
TC_TOK = TC_SEQS * SEQ


def _sc_gather_body(tok_hbm, x_hbm, out_hbm, idx_v, rows, gsems, ssems):
    wid = lax.axis_index("s") * 2 + lax.axis_index("c")
    tok_base = wid * TOK_PER_W

    pltpu.sync_copy(x_hbm.at[pl.ds(tok_base, TOK_PER_W)], idx_v)

    def start_gather(g, b):
        pltpu.async_copy(tok_hbm.at[idx_v.at[pl.ds(g * CHUNK, CHUNK)]],
                         rows[b], gsems[b])

    def wait_gather(g, b):
        pltpu.make_async_copy(tok_hbm.at[idx_v.at[pl.ds(g * CHUNK, CHUNK)]],
                              rows[b], gsems[b]).wait()

    def start_store(g, b):
        pltpu.async_copy(rows[b], out_hbm.at[pl.ds(tok_base + g * CHUNK, CHUNK)],
                         ssems[b])

    def wait_store(g, b):
        pltpu.make_async_copy(
            rows[b], out_hbm.at[pl.ds(tok_base + g * CHUNK, CHUNK)],
            ssems[b]).wait()

    for b in range(2):
        start_gather(b, b)

    def superchunk(p, _):
        for b in range(NBUF):
            g = p * NBUF + b

            @pl.when(g >= 2)
            def _():
                wait_store(g - 2, (b + 2) % NBUF)

            @pl.when(g + 2 < NCHUNKS)
            def _():
                start_gather(g + 2, (b + 2) % NBUF)

            wait_gather(g, b)
            start_store(g, b)
        return 0

    lax.fori_loop(0, NCHUNKS // NBUF, superchunk, 0, unroll=False)
    for g in range((NCHUNKS // NBUF) * NBUF, NCHUNKS):
        wait_store(g - 2, (g - 2) % NBUF)
        wait_gather(g, g % NBUF)
        start_store(g, g % NBUF)
    for g in (NCHUNKS - 2, NCHUNKS - 1):
        wait_store(g, g % NBUF)


def _sc_gather(tok_pk, x_slice):
    mesh = plsc.VectorSubcoreMesh(core_axis_name="c", subcore_axis_name="s")
    return pl.kernel(
        _sc_gather_body,
        out_type=jax.ShapeDtypeStruct((SLICE_TOK, H), jnp.int32),
        mesh=mesh,
        compiler_params=pltpu.CompilerParams(needs_layout_passes=False,
                                             use_tc_tiling_on_sc=False),
        scratch_types=[
            pltpu.VMEM((TOK_PER_W,), jnp.int32),
            [pltpu.VMEM((CHUNK, H), jnp.int32)] * NBUF,
            [pltpu.SemaphoreType.DMA] * NBUF,
            [pltpu.SemaphoreType.DMA] * NBUF,
        ],
    )(tok_pk, x_slice)


def _tc_ln_kernel(w_ref, pos_ref, gamma_ref, beta_ref, out_ref):
    w = w_ref[...]
    himask = jnp.full(w.shape, -65536, jnp.int32)
    lo = lax.bitcast_convert_type(lax.shift_left(w, 16), jnp.float32)
    hi = lax.bitcast_convert_type(jnp.bitwise_and(w, himask), jnp.float32)
    lo = lo.reshape(TC_SEQS, SEQ, H) + pos_ref[:, :H][None]
    hi = hi.reshape(TC_SEQS, SEQ, H) + pos_ref[:, H:][None]
    s = (jnp.sum(lo, axis=-1, keepdims=True)
         + jnp.sum(hi, axis=-1, keepdims=True))
    q = (jnp.sum(lo * lo, axis=-1, keepdims=True)
         + jnp.sum(hi * hi, axis=-1, keepdims=True))
    mean = s * (1.0 / D)
    var = q * (1.0 / D) - mean * mean
    rstd = lax.rsqrt(var + 1e-5)
    nlo = (lo - mean) * rstd * gamma_ref[0, :H] + beta_ref[0, :H]
    nhi = (hi - mean) * rstd * gamma_ref[0, H:] + beta_ref[0, H:]
    out_ref[:, :H] = nlo.reshape(TC_TOK, H)
    out_ref[:, H:] = nhi.reshape(TC_TOK, H)


def _tc_ln(gathered, pos_table, gamma2, beta2):
    grid = SLICE_TOK // TC_TOK
    return pl.pallas_call(
        _tc_ln_kernel,
        grid=(grid,),
        in_specs=[
            pl.BlockSpec((TC_TOK, H), lambda i: (i, 0)),
            pl.BlockSpec((SEQ, D), lambda i: (0, 0)),
            pl.BlockSpec((1, D), lambda i: (0, 0)),
            pl.BlockSpec((1, D), lambda i: (0, 0)),
        ],
        out_specs=pl.BlockSpec((TC_TOK, D), lambda i: (i, 0)),
        out_shape=jax.ShapeDtypeStruct((SLICE_TOK, D), jnp.float32),
    )(gathered, pos_table, gamma2, beta2)


@functools.partial(jax.jit, static_argnames=())
def kernel(x, tok_table, pos_table, gamma, beta):
    x_flat = x.astype(jnp.int32).reshape(BATCH * SEQ)
    tok_bf = tok_table.astype(jnp.bfloat16)
    tok_pk = lax.bitcast_convert_type(
        jnp.stack([tok_bf[:, :H], tok_bf[:, H:]], axis=-1), jnp.int32)
    gamma2 = gamma.reshape(1, D)
    beta2 = beta.reshape(1, D)
    outs = []
    for k in range(K_SLICES):
        xs = lax.dynamic_slice_in_dim(x_flat, k * SLICE_TOK, SLICE_TOK)
        g = _sc_gather(tok_pk, xs)
        outs.append(_tc_ln(g, pos_table, gamma2, beta2))
    return jnp.concatenate(outs, axis=0).reshape(BATCH, SEQ, D)
